# baseline (speedup 1.0000x reference)
import functools

import jax
import jax.numpy as jnp
from jax import lax
from jax.experimental import pallas as pl
from jax.experimental.pallas import tpu as pltpu
from jax.experimental.pallas import tpu_sc as plsc

B = 4096
L = 200
F = 64
NC = 2
NS = 16
NW = NC * NS
ROWS_PER_TILE = B // NW
LANES = 16
FCHUNKS = F // LANES


def _sc_pool_kernel(idxT_hbm, table_hbm, out_hbm, idx_v, rows_v, acc_v, sem0, sem1):
  wid = lax.axis_index("s") * NC + lax.axis_index("c")
  base = wid * ROWS_PER_TILE

  pltpu.sync_copy(idxT_hbm.at[:, pl.ds(base, ROWS_PER_TILE)], idx_v)

  @plsc.parallel_loop(0, ROWS_PER_TILE, step=1, unroll=8)
  def _(i):
    for k in range(FCHUNKS):
      acc_v[i, pl.ds(k * LANES, LANES)] = jnp.zeros((LANES,), jnp.float32)

  sems = (sem0, sem1)

  def start(j, buf):
    pltpu.async_copy(table_hbm.at[idx_v.at[j]], rows_v.at[buf], sems[buf])

  def wait(buf):
    pltpu.make_async_copy(
        table_hbm.at[idx_v.at[0]], rows_v.at[buf], sems[buf]).wait()

  def accum(buf):
    @plsc.parallel_loop(0, ROWS_PER_TILE, step=1, unroll=4)
    def _(i):
      for k in range(FCHUNKS):
        plsc.addupdate(acc_v.at[i, pl.ds(k * LANES, LANES)],
                       rows_v[buf, i, pl.ds(k * LANES, LANES)])

  start(0, 0)

  def outer(g, _):
    j0 = g * 2
    start(j0 + 1, 1)
    wait(0)
    accum(0)

    @pl.when(j0 + 2 < L)
    def _():
      start(j0 + 2, 0)

    wait(1)
    accum(1)
    return 0

  lax.fori_loop(0, L // 2, outer, 0)

  pltpu.sync_copy(acc_v, out_hbm.at[pl.ds(base, ROWS_PER_TILE)])


def _sc_pool(idxT, table):
  mesh = plsc.VectorSubcoreMesh(core_axis_name="c", subcore_axis_name="s")
  kern = pl.kernel(
      _sc_pool_kernel,
      out_type=jax.ShapeDtypeStruct((B, F), jnp.float32),
      mesh=mesh,
      scratch_types=[
          pltpu.VMEM((L, ROWS_PER_TILE), jnp.int32),
          pltpu.VMEM((2, ROWS_PER_TILE, F), jnp.float32),
          pltpu.VMEM((ROWS_PER_TILE, F), jnp.float32),
          pltpu.SemaphoreType.DMA,
          pltpu.SemaphoreType.DMA,
      ],
      compiler_params=pltpu.CompilerParams(use_tc_tiling_on_sc=False),
  )
  return kern(idxT, table)


def _tc_head_kernel(x_ref, w_ref, b_ref, bng_ref, bnb_ref, bnm_ref, bnv_ref,
                    lng_ref, lnb_ref, o_ref):
  x = x_ref[...] * (1.0 / L)
  y = jnp.dot(x, w_ref[...], preferred_element_type=jnp.float32) + b_ref[...]
  inv = lax.rsqrt(bnv_ref[...] + 1e-3)
  y = (y - bnm_ref[...]) * inv * bng_ref[...] + bnb_ref[...]
  mu = jnp.mean(y, axis=-1, keepdims=True)
  yc = y - mu
  var = jnp.mean(yc * yc, axis=-1, keepdims=True)
  o_ref[...] = yc * lax.rsqrt(var + 1e-3) * lng_ref[...] + lnb_ref[...]


def _tc_head(pooled, W, b, bn_gamma, bn_beta, bn_mean, bn_var, ln_gamma, ln_beta):
  blk = 512
  grid = B // blk
  vec_spec = pl.BlockSpec((1, F), lambda i: (0, 0))
  return pl.pallas_call(
      _tc_head_kernel,
      grid=(grid,),
      in_specs=[
          pl.BlockSpec((blk, F), lambda i: (i, 0)),
          pl.BlockSpec((F, F), lambda i: (0, 0)),
          vec_spec, vec_spec, vec_spec, vec_spec, vec_spec, vec_spec, vec_spec,
      ],
      out_specs=pl.BlockSpec((blk, F), lambda i: (i, 0)),
      out_shape=jax.ShapeDtypeStruct((B, F), jnp.float32),
  )(pooled, W, b.reshape(1, F), bn_gamma.reshape(1, F), bn_beta.reshape(1, F),
    bn_mean.reshape(1, F), bn_var.reshape(1, F), ln_gamma.reshape(1, F),
    ln_beta.reshape(1, F))


@jax.jit
def kernel(inputs, table, W, b, bn_gamma, bn_beta, bn_mean, bn_var, ln_gamma, ln_beta):
  idxT = inputs.astype(jnp.int32).T
  pooled = _sc_pool(idxT, table)
  return _tc_head(pooled, W, b, bn_gamma, bn_beta, bn_mean, bn_var,
                  ln_gamma, ln_beta)
